# baseline (speedup 1.0000x reference)
import functools

import jax
import jax.numpy as jnp
from jax.experimental import pallas as pl
from jax.experimental.pallas import tpu as pltpu

_LANES = 128


def _pool_body(x_ref, o_ref, *, n_full, tail, inv_l):
    if n_full == 0:
        s = jnp.sum(x_ref[...].astype(jnp.float32), axis=-1, keepdims=True)
    else:
        acc = x_ref[:, 0:_LANES].astype(jnp.float32)
        for c in range(1, n_full):
            acc = acc + x_ref[:, c * _LANES:(c + 1) * _LANES].astype(jnp.float32)
        s = jnp.sum(acc, axis=-1, keepdims=True)
        if tail:
            t = x_ref[:, n_full * _LANES:].astype(jnp.float32)
            s = s + jnp.sum(t, axis=-1, keepdims=True)
    o_ref[...] = (s * jnp.float32(inv_l)).astype(o_ref.dtype)


def _pick_tb(B, L, itemsize, target_bytes=4 << 20):
    row_bytes = L * itemsize
    tb = max(8, min(1024, (target_bytes // row_bytes) // 8 * 8))
    while tb > 8 and -(-B // tb) < 2:
        tb //= 2
    return tb


def kernel(x):
    shape = x.shape
    L = shape[-1]
    lead = shape[:-1]
    B = 1
    for d in lead:
        B *= d
    x2 = x.reshape(B, L)

    itemsize = jnp.dtype(x.dtype).itemsize
    TB = _pick_tb(B, L, itemsize)
    grid_b = -(-B // TB)

    n_full = L // _LANES
    tail = L % _LANES != 0

    out = pl.pallas_call(
        functools.partial(_pool_body, n_full=n_full, tail=tail, inv_l=1.0 / L),
        out_shape=jax.ShapeDtypeStruct((B, 1), x.dtype),
        grid=(grid_b,),
        in_specs=[pl.BlockSpec((TB, L), lambda b: (b, 0))],
        out_specs=pl.BlockSpec((TB, 1), lambda b: (b, 0)),
        compiler_params=pltpu.CompilerParams(
            dimension_semantics=("parallel",),
            vmem_limit_bytes=64 << 20),
    )(x2)

    return out.reshape(lead)


# speedup vs baseline: 1.1688x; 1.1688x over previous
import functools

import jax
import jax.numpy as jnp
from jax.experimental import pallas as pl
from jax.experimental.pallas import tpu as pltpu

_LANES = 128


def _pool_body(x_ref, o_ref, *, n_full, tail, inv_l):
    if n_full == 0:
        s = jnp.sum(x_ref[...].astype(jnp.float32), axis=-1, keepdims=True)
    else:
        acc = x_ref[:, 0:_LANES].astype(jnp.float32)
        for c in range(1, n_full):
            acc = acc + x_ref[:, c * _LANES:(c + 1) * _LANES].astype(jnp.float32)
        s = jnp.sum(acc, axis=-1, keepdims=True)
        if tail:
            t = x_ref[:, n_full * _LANES:].astype(jnp.float32)
            s = s + jnp.sum(t, axis=-1, keepdims=True)
    o_ref[...] = (s * jnp.float32(inv_l)).astype(o_ref.dtype)


def _pick_tb(B, L, itemsize, target_bytes=4 << 20):
    row_bytes = L * itemsize
    tb = max(8, min(1024, (target_bytes // row_bytes) // 8 * 8))
    while tb > 8 and -(-B // tb) < 2:
        tb //= 2
    return tb


def kernel(x):
    shape = x.shape
    L = shape[-1]
    lead = shape[:-1]
    B = 1
    for d in lead:
        B *= d
    x2 = x.reshape(B, L)

    itemsize = jnp.dtype(x.dtype).itemsize
    TB = _pick_tb(B, L, itemsize)
    grid_b = -(-B // TB)

    n_full = L // _LANES
    tail = L % _LANES != 0

    in_block = TB * L * itemsize
    vlim = int(min(max(4 * in_block + (4 << 20), 16 << 20), 48 << 20))
    out = pl.pallas_call(
        functools.partial(_pool_body, n_full=n_full, tail=tail, inv_l=1.0 / L),
        out_shape=jax.ShapeDtypeStruct((B, 1), x.dtype),
        grid=(grid_b,),
        in_specs=[pl.BlockSpec((TB, L), lambda b: (b, 0))],
        out_specs=pl.BlockSpec((TB, 1), lambda b: (b, 0)),
        compiler_params=pltpu.CompilerParams(
            dimension_semantics=("parallel",),
            vmem_limit_bytes=vlim),
    )(x2)

    return out.reshape(lead)


# speedup vs baseline: 1.2358x; 1.0573x over previous
import functools

import jax
import jax.numpy as jnp
from jax.experimental import pallas as pl
from jax.experimental.pallas import tpu as pltpu

_LANES = 128


def _pool_body(x_ref, o_ref, *, n_full, tail, inv_l):
    if n_full == 0:
        s = jnp.sum(x_ref[...].astype(jnp.float32), axis=-1, keepdims=True)
    else:
        acc = x_ref[:, 0:_LANES].astype(jnp.float32)
        for c in range(1, n_full):
            acc = acc + x_ref[:, c * _LANES:(c + 1) * _LANES].astype(jnp.float32)
        s = jnp.sum(acc, axis=-1, keepdims=True)
        if tail:
            t = x_ref[:, n_full * _LANES:].astype(jnp.float32)
            s = s + jnp.sum(t, axis=-1, keepdims=True)
    o_ref[...] = (s * jnp.float32(inv_l)).astype(o_ref.dtype)


def _pick_tb(B, L, itemsize, target_bytes=8 << 20):
    row_bytes = L * itemsize
    tb = max(8, min(1024, (target_bytes // row_bytes) // 8 * 8))
    while tb > 8 and -(-B // tb) < 2:
        tb //= 2
    return tb


def kernel(x):
    shape = x.shape
    L = shape[-1]
    lead = shape[:-1]
    B = 1
    for d in lead:
        B *= d
    x2 = x.reshape(B, L)

    itemsize = jnp.dtype(x.dtype).itemsize
    TB = _pick_tb(B, L, itemsize)
    grid_b = -(-B // TB)

    n_full = L // _LANES
    tail = L % _LANES != 0

    in_block = TB * L * itemsize
    vlim = int(min(max(4 * in_block + (4 << 20), 16 << 20), 48 << 20))
    out = pl.pallas_call(
        functools.partial(_pool_body, n_full=n_full, tail=tail, inv_l=1.0 / L),
        out_shape=jax.ShapeDtypeStruct((B, 1), x.dtype),
        grid=(grid_b,),
        in_specs=[pl.BlockSpec((TB, L), lambda b: (b, 0))],
        out_specs=pl.BlockSpec((TB, 1), lambda b: (b, 0)),
        compiler_params=pltpu.CompilerParams(
            dimension_semantics=("parallel",),
            vmem_limit_bytes=vlim),
    )(x2)

    return out.reshape(lead)
